# baseline (speedup 1.0000x reference)
import functools

import jax
import jax.numpy as jnp
from jax import lax
from jax.experimental import pallas as pl
from jax.experimental.pallas import tpu as pltpu
from jax.experimental.pallas import tpu_sc as plsc

_B = 4096
_D = 64
_C = 100
_NW = 32
_BPW = _B // _NW
_R = 112


def _dpos_sc_body(feats_hbm, labels_hbm, centers_hbm, out_hbm,
                  idx_v, f_v, tab_v, out_v, fsem, tsem):
    wid = lax.axis_index("s") * 2 + lax.axis_index("c")
    base = wid * _BPW
    pltpu.sync_copy(labels_hbm.at[pl.ds(base, _BPW)], idx_v)
    fcopy = pltpu.async_copy(feats_hbm.at[pl.ds(base, _BPW)], f_v, fsem)
    tcopy = pltpu.async_copy(centers_hbm, tab_v, tsem)
    fcopy.wait()
    tcopy.wait()

    lane = lax.iota(jnp.int32, 16)
    for g in range(_BPW // 16):
        labs = idx_v[pl.ds(g * 16, 16)]
        res = jnp.zeros((16,), jnp.float32)
        for t in range(16):
            i = g * 16 + t
            l = labs[t]
            acc = None
            for k in range(_D // 16):
                df = f_v[i, pl.ds(k * 16, 16)] - tab_v[l, pl.ds(k * 16, 16)]
                sq = df * df
                acc = sq if acc is None else acc + sq
            s = None
            for k in range(16):
                v = acc[k]
                s = v if s is None else s + v
            res = jnp.where(lane == t, s, res)
        out_v[pl.ds(g * 16, 16)] = res

    pltpu.sync_copy(out_v, out_hbm.at[pl.ds(base, _BPW)])


def _dpos_sc(features, labels, centers_pad):
    k = functools.partial(
        pl.kernel,
        mesh=plsc.VectorSubcoreMesh(core_axis_name="c", subcore_axis_name="s"),
        out_type=jax.ShapeDtypeStruct((_B,), jnp.float32),
        scratch_types=[
            pltpu.VMEM((_BPW,), jnp.int32),
            pltpu.VMEM((_BPW, _D), jnp.float32),
            pltpu.VMEM((_C, 2 * _D), jnp.float32),
            pltpu.VMEM((_BPW,), jnp.float32),
            pltpu.SemaphoreType.DMA,
            pltpu.SemaphoreType.DMA,
        ],
    )(_dpos_sc_body)
    return k(features, labels, centers_pad)


def _dneg_body(feats_ref, labels_ref, centers_ref, out_ref):
    f = feats_ref[...]
    ft = f.T
    fct = jnp.concatenate([ft, ft * ft], axis=0)

    c = centers_ref[...]
    czero = jnp.concatenate(
        [c, jnp.zeros((_C, _D), jnp.float32)], axis=1)
    row0 = jnp.concatenate([jnp.zeros((1, _D), jnp.float32),
                            jnp.ones((1, _D), jnp.float32)], axis=1)
    top = jnp.concatenate([row0, jnp.zeros((7, 2 * _D), jnp.float32)], axis=0)
    bottom = jnp.zeros((_R - 8 - _C, 2 * _D), jnp.float32)
    caug = jnp.concatenate([top, czero, bottom], axis=0)

    g = jnp.dot(caug, fct, preferred_element_type=jnp.float32)
    fn = g[0:1, :]
    cn = jnp.sum(caug * caug, axis=1, keepdims=True)

    labels = labels_ref[...]
    row = jax.lax.broadcasted_iota(jnp.int32, g.shape, 0)
    valid = (row >= 8) & (row < 8 + _C) & (row != labels + 8)
    big = jnp.float32(3.0e38)
    m = jnp.min(jnp.where(valid, cn - 2.0 * g, big), axis=0, keepdims=True)
    out_ref[...] = jnp.maximum(fn + m, 0.0)


def _loss_body(dpos_ref, dneg_ref, margin_ref, out_ref):
    d_pos = jnp.sqrt(dpos_ref[...])
    d_neg = jnp.sqrt(dneg_ref[...])
    margin = margin_ref[0, 0]
    per_row = jnp.maximum(d_pos + margin - d_neg, 0.0)
    out_ref[0, 0] = jnp.sum(per_row) / _B


def kernel(features, labels, margin, centers):
    labels_lane = labels.reshape(1, _B)
    margin_arr = jnp.asarray(margin, jnp.float32).reshape(1, 1)

    centers_pad = jnp.pad(centers, ((0, 0), (0, _D)))
    dpos2 = _dpos_sc(features, labels, centers_pad)

    dneg2 = pl.pallas_call(
        _dneg_body,
        out_shape=jax.ShapeDtypeStruct((1, _B), jnp.float32),
        in_specs=[
            pl.BlockSpec(memory_space=pltpu.VMEM),
            pl.BlockSpec(memory_space=pltpu.VMEM),
            pl.BlockSpec(memory_space=pltpu.VMEM),
        ],
        out_specs=pl.BlockSpec(memory_space=pltpu.VMEM),
    )(features, labels_lane, centers)

    out = pl.pallas_call(
        _loss_body,
        out_shape=jax.ShapeDtypeStruct((1, 1), jnp.float32),
        in_specs=[
            pl.BlockSpec(memory_space=pltpu.VMEM),
            pl.BlockSpec(memory_space=pltpu.VMEM),
            pl.BlockSpec(memory_space=pltpu.SMEM),
        ],
        out_specs=pl.BlockSpec(memory_space=pltpu.SMEM),
    )(dpos2.reshape(1, _B), dneg2, margin_arr)
    return out[0, 0]


# speedup vs baseline: 1.5653x; 1.5653x over previous
import functools

import jax
import jax.numpy as jnp
from jax import lax
from jax.experimental import pallas as pl
from jax.experimental.pallas import tpu as pltpu
from jax.experimental.pallas import tpu_sc as plsc

_B = 4096
_D = 64
_C = 100
_NW = 32
_BPW = _B // _NW
_R = 112


def _dpos_sc_body(feats_hbm, labels_hbm, centers_hbm, out_hbm,
                  idx_v, f_v, tab_v, acc_v, out_v, fsem, tsem):
    wid = lax.axis_index("s") * 2 + lax.axis_index("c")
    base = wid * _BPW
    pltpu.sync_copy(labels_hbm.at[pl.ds(base, _BPW)], idx_v)
    fcopy = pltpu.async_copy(feats_hbm.at[pl.ds(base, _BPW)], f_v, fsem)
    tcopy = pltpu.async_copy(centers_hbm, tab_v, tsem)
    fcopy.wait()
    tcopy.wait()

    lane = lax.iota(jnp.int32, 16)
    for g in range(_BPW // 16):
        labs = idx_v[pl.ds(g * 16, 16)]
        for t in range(16):
            i = g * 16 + t
            l = labs[t]
            acc = None
            for k in range(_D // 16):
                df = f_v[i, pl.ds(k * 16, 16)] - tab_v[l, pl.ds(k * 16, 16)]
                sq = df * df
                acc = sq if acc is None else acc + sq
            acc_v[t, :] = acc
        res = None
        for k in range(16):
            col = plsc.load_gather(
                acc_v, [lane, jnp.full((16,), k, jnp.int32)])
            res = col if res is None else res + col
        out_v[pl.ds(g * 16, 16)] = res

    pltpu.sync_copy(out_v, out_hbm.at[pl.ds(base, _BPW)])


def _dpos_sc(features, labels, centers_pad):
    k = functools.partial(
        pl.kernel,
        mesh=plsc.VectorSubcoreMesh(core_axis_name="c", subcore_axis_name="s"),
        compiler_params=pltpu.CompilerParams(needs_layout_passes=False),
        out_type=jax.ShapeDtypeStruct((_B,), jnp.float32),
        scratch_types=[
            pltpu.VMEM((_BPW,), jnp.int32),
            pltpu.VMEM((_BPW, _D), jnp.float32),
            pltpu.VMEM((_C, 2 * _D), jnp.float32),
            pltpu.VMEM((16, 16), jnp.float32),
            pltpu.VMEM((_BPW,), jnp.float32),
            pltpu.SemaphoreType.DMA,
            pltpu.SemaphoreType.DMA,
        ],
    )(_dpos_sc_body)
    return k(features, labels, centers_pad)


def _dneg_body(feats_ref, labels_ref, centers_ref, out_ref):
    f = feats_ref[...]
    ft = f.T
    fct = jnp.concatenate([ft, ft * ft], axis=0)

    c = centers_ref[...]
    czero = jnp.concatenate(
        [c, jnp.zeros((_C, _D), jnp.float32)], axis=1)
    row0 = jnp.concatenate([jnp.zeros((1, _D), jnp.float32),
                            jnp.ones((1, _D), jnp.float32)], axis=1)
    top = jnp.concatenate([row0, jnp.zeros((7, 2 * _D), jnp.float32)], axis=0)
    bottom = jnp.zeros((_R - 8 - _C, 2 * _D), jnp.float32)
    caug = jnp.concatenate([top, czero, bottom], axis=0)

    g = jnp.dot(caug, fct, preferred_element_type=jnp.float32)
    fn = g[0:1, :]
    cn = jnp.sum(caug * caug, axis=1, keepdims=True)

    labels = labels_ref[...]
    row = jax.lax.broadcasted_iota(jnp.int32, g.shape, 0)
    valid = (row >= 8) & (row < 8 + _C) & (row != labels + 8)
    big = jnp.float32(3.0e38)
    m = jnp.min(jnp.where(valid, cn - 2.0 * g, big), axis=0, keepdims=True)
    out_ref[...] = jnp.maximum(fn + m, 0.0)


def _loss_body(dpos_ref, dneg_ref, margin_ref, out_ref):
    d_pos = jnp.sqrt(dpos_ref[...])
    d_neg = jnp.sqrt(dneg_ref[...])
    margin = margin_ref[0, 0]
    per_row = jnp.maximum(d_pos + margin - d_neg, 0.0)
    out_ref[0, 0] = jnp.sum(per_row) / _B


def kernel(features, labels, margin, centers):
    labels_lane = labels.reshape(1, _B)
    margin_arr = jnp.asarray(margin, jnp.float32).reshape(1, 1)

    centers_pad = jnp.pad(centers, ((0, 0), (0, _D)))
    dpos2 = _dpos_sc(features, labels, centers_pad)

    dneg2 = pl.pallas_call(
        _dneg_body,
        out_shape=jax.ShapeDtypeStruct((1, _B), jnp.float32),
        in_specs=[
            pl.BlockSpec(memory_space=pltpu.VMEM),
            pl.BlockSpec(memory_space=pltpu.VMEM),
            pl.BlockSpec(memory_space=pltpu.VMEM),
        ],
        out_specs=pl.BlockSpec(memory_space=pltpu.VMEM),
    )(features, labels_lane, centers)

    out = pl.pallas_call(
        _loss_body,
        out_shape=jax.ShapeDtypeStruct((1, 1), jnp.float32),
        in_specs=[
            pl.BlockSpec(memory_space=pltpu.VMEM),
            pl.BlockSpec(memory_space=pltpu.VMEM),
            pl.BlockSpec(memory_space=pltpu.SMEM),
        ],
        out_specs=pl.BlockSpec(memory_space=pltpu.SMEM),
    )(dpos2.reshape(1, _B), dneg2, margin_arr)
    return out[0, 0]


# speedup vs baseline: 1.5923x; 1.0173x over previous
import functools

import jax
import jax.numpy as jnp
from jax import lax
from jax.experimental import pallas as pl
from jax.experimental.pallas import tpu as pltpu
from jax.experimental.pallas import tpu_sc as plsc

_B = 4096
_D = 64
_C = 100
_NW = 32
_BPW = _B // _NW
_R = 112


def _dpos_sc_body(feats_hbm, labels_hbm, centers_hbm, out_hbm,
                  idx_v, f_v, rows_v, acc_v, out_v, fsem, tsem):
    wid = lax.axis_index("s") * 2 + lax.axis_index("c")
    base = wid * _BPW
    pltpu.sync_copy(labels_hbm.at[pl.ds(base, _BPW)], idx_v)
    fcopy = pltpu.async_copy(feats_hbm.at[pl.ds(base, _BPW)], f_v, fsem)
    gcopy = pltpu.async_copy(centers_hbm.at[idx_v], rows_v, tsem)
    fcopy.wait()
    gcopy.wait()

    lane = lax.iota(jnp.int32, 16)
    for g in range(_BPW // 16):
        for t in range(16):
            i = g * 16 + t
            acc = None
            for k in range(_D // 16):
                df = f_v[i, pl.ds(k * 16, 16)] - rows_v[i, pl.ds(k * 16, 16)]
                sq = df * df
                acc = sq if acc is None else acc + sq
            acc_v[t, :] = acc
        res = None
        for k in range(16):
            col = plsc.load_gather(
                acc_v, [lane, jnp.full((16,), k, jnp.int32)])
            res = col if res is None else res + col
        out_v[pl.ds(g * 16, 16)] = res

    pltpu.sync_copy(out_v, out_hbm.at[pl.ds(base, _BPW)])


def _dpos_sc(features, labels, centers_pad):
    k = functools.partial(
        pl.kernel,
        mesh=plsc.VectorSubcoreMesh(core_axis_name="c", subcore_axis_name="s"),
        compiler_params=pltpu.CompilerParams(needs_layout_passes=False),
        out_type=jax.ShapeDtypeStruct((_B,), jnp.float32),
        scratch_types=[
            pltpu.VMEM((_BPW,), jnp.int32),
            pltpu.VMEM((_BPW, _D), jnp.float32),
            pltpu.VMEM((_BPW, 2 * _D), jnp.float32),
            pltpu.VMEM((16, 16), jnp.float32),
            pltpu.VMEM((_BPW,), jnp.float32),
            pltpu.SemaphoreType.DMA,
            pltpu.SemaphoreType.DMA,
        ],
    )(_dpos_sc_body)
    return k(features, labels, centers_pad)


def _dneg_body(feats_ref, labels_ref, centers_ref, out_ref):
    f = feats_ref[...]
    ft = f.T
    fct = jnp.concatenate([ft, ft * ft], axis=0)

    c = centers_ref[...]
    czero = jnp.concatenate(
        [c, jnp.zeros((_C, _D), jnp.float32)], axis=1)
    row0 = jnp.concatenate([jnp.zeros((1, _D), jnp.float32),
                            jnp.ones((1, _D), jnp.float32)], axis=1)
    top = jnp.concatenate([row0, jnp.zeros((7, 2 * _D), jnp.float32)], axis=0)
    bottom = jnp.zeros((_R - 8 - _C, 2 * _D), jnp.float32)
    caug = jnp.concatenate([top, czero, bottom], axis=0)

    g = jnp.dot(caug, fct, preferred_element_type=jnp.float32)
    fn = g[0:1, :]
    cn = jnp.sum(caug * caug, axis=1, keepdims=True)

    labels = labels_ref[...]
    row = jax.lax.broadcasted_iota(jnp.int32, g.shape, 0)
    valid = (row >= 8) & (row < 8 + _C) & (row != labels + 8)
    big = jnp.float32(3.0e38)
    m = jnp.min(jnp.where(valid, cn - 2.0 * g, big), axis=0, keepdims=True)
    out_ref[...] = jnp.maximum(fn + m, 0.0)


def _loss_body(dpos_ref, dneg_ref, margin_ref, out_ref):
    d_pos = jnp.sqrt(dpos_ref[...])
    d_neg = jnp.sqrt(dneg_ref[...])
    margin = margin_ref[0, 0]
    per_row = jnp.maximum(d_pos + margin - d_neg, 0.0)
    out_ref[0, 0] = jnp.sum(per_row) / _B


def kernel(features, labels, margin, centers):
    labels_lane = labels.reshape(1, _B)
    margin_arr = jnp.asarray(margin, jnp.float32).reshape(1, 1)

    centers_pad = jnp.pad(centers, ((0, 0), (0, _D)))
    dpos2 = _dpos_sc(features, labels, centers_pad)

    dneg2 = pl.pallas_call(
        _dneg_body,
        out_shape=jax.ShapeDtypeStruct((1, _B), jnp.float32),
        in_specs=[
            pl.BlockSpec(memory_space=pltpu.VMEM),
            pl.BlockSpec(memory_space=pltpu.VMEM),
            pl.BlockSpec(memory_space=pltpu.VMEM),
        ],
        out_specs=pl.BlockSpec(memory_space=pltpu.VMEM),
    )(features, labels_lane, centers)

    out = pl.pallas_call(
        _loss_body,
        out_shape=jax.ShapeDtypeStruct((1, 1), jnp.float32),
        in_specs=[
            pl.BlockSpec(memory_space=pltpu.VMEM),
            pl.BlockSpec(memory_space=pltpu.VMEM),
            pl.BlockSpec(memory_space=pltpu.SMEM),
        ],
        out_specs=pl.BlockSpec(memory_space=pltpu.SMEM),
    )(dpos2.reshape(1, _B), dneg2, margin_arr)
    return out[0, 0]
